# baseline (speedup 1.0000x reference)
import functools

import jax
import jax.numpy as jnp
from jax import lax
from jax.experimental import pallas as pl
from jax.experimental.pallas import tpu as pltpu
from jax.experimental.pallas import tpu_sc as plsc

NC = 2
NS = 16
NW = NC * NS
L = 16
C = 16


def _emb_body(x_hbm, tok_hbm, pos_hbm, out_hbm, idx_v, pos_v, rows_v, gsem):
    Bn, n_chunks_t, _ = x_hbm.shape
    D = tok_hbm.shape[1]
    per_w = n_chunks_t // NW
    wid = lax.axis_index("s") * NC + lax.axis_index("c")
    c0 = wid * per_w

    for b in range(Bn):
        pltpu.sync_copy(x_hbm.at[b, pl.ds(c0, per_w)], idx_v.at[b])

    def chunk_body(pc, _):
        t0 = (c0 + pc) * C
        pltpu.sync_copy(pos_hbm.at[pl.ds(t0, C)], pos_v)
        for b in range(Bn):
            pltpu.async_copy(tok_hbm.at[idx_v.at[b, pc]], rows_v, gsem).wait()

            def add_body(r, _):
                for k in range(D // L):
                    sl = pl.ds(k * L, L)
                    rows_v[r, sl] = rows_v[r, sl] + pos_v[r, sl]
                return 0

            lax.fori_loop(0, C, add_body, 0)
            pltpu.sync_copy(rows_v, out_hbm.at[b, pl.ds(t0, C)])
        return 0

    lax.fori_loop(0, per_w, chunk_body, 0)


def kernel(x, tok_emb, pos_emb):
    Bn, T = x.shape
    D = tok_emb.shape[1]
    x3 = x.reshape(Bn, T // C, C)
    per_w = (T // C) // NW

    call = functools.partial(
        pl.kernel,
        out_type=jax.ShapeDtypeStruct((Bn, T, D), jnp.float32),
        mesh=plsc.VectorSubcoreMesh(
            core_axis_name="c", subcore_axis_name="s",
            num_cores=NC, num_subcores=NS),
        scratch_types=[
            pltpu.VMEM((Bn, per_w, C), jnp.int32),
            pltpu.VMEM((C, D), jnp.float32),
            pltpu.VMEM((C, D), jnp.float32),
            pltpu.SemaphoreType.DMA,
        ],
    )(_emb_body)
    return call(x3, tok_emb, pos_emb[:T])


# speedup vs baseline: 1.3008x; 1.3008x over previous
import functools

import jax
import jax.numpy as jnp
from jax import lax
from jax.experimental import pallas as pl
from jax.experimental.pallas import tpu as pltpu
from jax.experimental.pallas import tpu_sc as plsc

NC = 2
NS = 16
NW = NC * NS
L = 16
C = 16
SLOTS = 2


def _emb_body(x_hbm, tok_hbm, pos_hbm, out_hbm, idx_v, pos_v, rows_v, sum_v,
              gsem, ssem):
    Bn, n_chunks_t, _ = x_hbm.shape
    D = tok_hbm.shape[1]
    per_w = n_chunks_t // NW
    n_it = per_w * Bn
    wid = lax.axis_index("s") * NC + lax.axis_index("c")
    c0 = wid * per_w

    for b in range(Bn):
        pltpu.sync_copy(x_hbm.at[b, pl.ds(c0, per_w)], idx_v.at[b])

    def gather_start(j, slot):
        pc = j // Bn
        b = lax.rem(j, Bn)
        pltpu.make_async_copy(
            tok_hbm.at[idx_v.at[b, pc]], rows_v.at[slot], gsem.at[slot]
        ).start()

    def gather_wait(slot):
        pltpu.make_async_copy(
            tok_hbm.at[idx_v.at[0, 0]], rows_v.at[slot], gsem.at[slot]
        ).wait()

    def store_start(j, slot):
        pc = j // Bn
        b = lax.rem(j, Bn)
        t0 = (c0 + pc) * C
        pltpu.make_async_copy(
            sum_v.at[slot], out_hbm.at[b, pl.ds(t0, C)], ssem.at[slot]
        ).start()

    def store_wait(slot):
        pltpu.make_async_copy(
            sum_v.at[slot], out_hbm.at[0, pl.ds(0, C)], ssem.at[slot]
        ).wait()

    def add_chunk(slot):
        def body(r, _):
            for kk in range(D // L):
                sl = pl.ds(kk * L, L)
                sum_v[slot, r, sl] = rows_v[slot, r, sl] + pos_v[r, sl]
            return 0
        lax.fori_loop(0, C, body, 0)

    gather_start(0, 0)
    gather_start(1, 1)

    @pl.loop(0, n_it, step=SLOTS)
    def _(i):
        for slot in range(SLOTS):
            j = i + slot
            pc = j // Bn
            b = lax.rem(j, Bn)

            @pl.when(b == 0)
            def _():
                pltpu.sync_copy(pos_hbm.at[pl.ds((c0 + pc) * C, C)], pos_v)

            gather_wait(slot)

            @pl.when(j >= SLOTS)
            def _():
                store_wait(slot)

            add_chunk(slot)

            @pl.when(j + SLOTS < n_it)
            def _():
                gather_start(j + SLOTS, slot)

            store_start(j, slot)

    store_wait(0)
    store_wait(1)


def kernel(x, tok_emb, pos_emb):
    Bn, T = x.shape
    D = tok_emb.shape[1]
    x3 = x.reshape(Bn, T // C, C)
    per_w = (T // C) // NW

    call = functools.partial(
        pl.kernel,
        out_type=jax.ShapeDtypeStruct((Bn, T, D), jnp.float32),
        mesh=plsc.VectorSubcoreMesh(
            core_axis_name="c", subcore_axis_name="s",
            num_cores=NC, num_subcores=NS),
        scratch_types=[
            pltpu.VMEM((Bn, per_w, C), jnp.int32),
            pltpu.VMEM((C, D), jnp.float32),
            pltpu.VMEM((SLOTS, C, D), jnp.float32),
            pltpu.VMEM((SLOTS, C, D), jnp.float32),
            pltpu.SemaphoreType.DMA((SLOTS,)),
            pltpu.SemaphoreType.DMA((SLOTS,)),
        ],
    )(_emb_body)
    return call(x3, tok_emb, pos_emb[:T])
